# baseline (speedup 1.0000x reference)
import functools

import jax
import jax.numpy as jnp
from jax import lax
from jax.experimental import pallas as pl
from jax.experimental.pallas import tpu as pltpu
from jax.experimental.pallas import tpu_sc as plsc

_D = 3
_B = 512
_IN = 512
_H1, _H2 = 512, 256
_G = 128
_NH = 4
_K = 10
_ALPHA = 0.2
_MARGIN = 1.0
_NCLS = 10
_N = (_D + 1) * _B
_F = _NH * _G
_RB = 256
_NEG = -9e15


def _mlp_body(x_ref, w0_ref, b0_ref, w1_ref, b1_ref, wg_ref, a_ref,
              o_ref, h_ref, f_ref):
    h = jnp.dot(x_ref[...], w0_ref[...], preferred_element_type=jnp.float32)
    h = jnp.maximum(h + b0_ref[...], 0.0)
    h = jnp.dot(h, w1_ref[...], preferred_element_type=jnp.float32)
    h2 = jnp.maximum(h + b1_ref[...], 0.0)
    o_ref[...] = h2
    ha = jnp.dot(h2, wg_ref[...], preferred_element_type=jnp.float32)
    h_ref[...] = ha
    f_ref[...] = jnp.dot(ha, a_ref[...], preferred_element_type=jnp.float32)


def _topk_body(hb_ref, hf_ref, idx_ref):
    i = pl.program_id(0)
    hb = hb_ref[...]
    hf = hf_ref[...]
    xxb = jnp.sum(hb * hb, axis=1, keepdims=True)
    xxf = jnp.sum(hf * hf, axis=1)
    g = jax.lax.dot_general(hb, hf, (((1,), (1,)), ((), ())),
                            preferred_element_type=jnp.float32)
    d2 = jnp.maximum(xxb + xxf[None, :] - 2.0 * g, 0.0)
    cols = jax.lax.broadcasted_iota(jnp.int32, d2.shape, 1)
    rows = jax.lax.broadcasted_iota(jnp.int32, d2.shape, 0) + i * _RB
    neg = jnp.where(rows == cols, -1e12, -d2)
    picks = []
    for _ in range(_K):
        m = jnp.max(neg, axis=1, keepdims=True)
        it = jnp.min(jnp.where(neg == m, cols, _N), axis=1, keepdims=True)
        picks.append(it)
        neg = jnp.where(cols == it, -jnp.float32(jnp.inf), neg)
    picks.append(jnp.zeros((_RB, 16 - _K), jnp.int32))
    idx_ref[...] = jnp.concatenate(picks, axis=1)


def _hff_body(x_ref, w_ref, a_ref, h_ref, f_ref):
    h = jnp.dot(x_ref[...], w_ref[...], preferred_element_type=jnp.float32)
    h_ref[...] = h
    f_ref[...] = jnp.dot(h, a_ref[...], preferred_element_type=jnp.float32)


def _log_softmax(x):
    m = jnp.max(x, axis=1, keepdims=True)
    return (x - m) - jnp.log(jnp.sum(jnp.exp(x - m), axis=1, keepdims=True))


def _heads_body(x_ref, lab_ref, labt_ref, wc_ref, bc_ref, wd_ref, bd_ref,
                lp_ref, sd_ref, td_ref, tl_ref):
    gt = x_ref[_D * _B:, :]
    rt = jnp.maximum(gt, 0.0)
    cols = jax.lax.broadcasted_iota(jnp.int32, (_B, _B), 1)
    rows = jax.lax.broadcasted_iota(jnp.int32, (_B, _B), 0)
    eye = rows == cols
    for d in range(_D):
        g = x_ref[d * _B:(d + 1) * _B, :]
        r = jnp.maximum(g, 0.0)
        lg = jnp.dot(r, wc_ref[...], preferred_element_type=jnp.float32) + bc_ref[...]
        lp_ref[d] = _log_softmax(lg)
        wd = wd_ref[d]
        bd = bd_ref[d:d + 1, :]
        sd_ref[d] = _log_softmax(
            jnp.dot(r, wd, preferred_element_type=jnp.float32) + bd)
        td_ref[d] = _log_softmax(
            jnp.dot(rt, wd, preferred_element_type=jnp.float32) + bd)
        nrm = jnp.sqrt(jnp.sum(g * g, axis=1, keepdims=True))
        gn = g / jnp.maximum(nrm, 1e-12)
        gram = jax.lax.dot_general(gn, gn, (((1,), (1,)), ((), ())),
                                   preferred_element_type=jnp.float32)
        diag = jnp.where(eye, gram, 0.0)
        xxc = jnp.sum(diag, axis=1, keepdims=True)
        xxr = jnp.sum(diag, axis=0, keepdims=True)
        dist = jnp.sqrt(jnp.maximum(xxc + xxr - 2.0 * gram, 0.0) + 1e-12)
        lr = lab_ref[d:d + 1, :]
        lc = labt_ref[:, d:d + 1]
        same = lc == lr
        pos_mask = same & (~eye)
        neg_mask = ~same
        pv = jnp.where(pos_mask, dist, -1.0)
        pm = jnp.max(pv, axis=1, keepdims=True)
        pidx = jnp.min(jnp.where(pv == pm, cols, _B), axis=1, keepdims=True)
        nv = jnp.where(neg_mask, dist, 1e12)
        nm = jnp.min(nv, axis=1, keepdims=True)
        nidx = jnp.min(jnp.where(nv == nm, cols, _B), axis=1, keepdims=True)
        pos_d = jnp.sum(jnp.where(cols == pidx, dist, 0.0), axis=1)
        neg_d = jnp.sum(jnp.where(cols == nidx, dist, 0.0), axis=1)
        hard = (neg_d - pos_d < _MARGIN).astype(jnp.float32)
        hinge = jnp.maximum(_MARGIN + pos_d - neg_d, 0.0)
        loss = jnp.sum(hinge * hard) / jnp.maximum(jnp.sum(hard), 1.0)
        tl_ref[d:d + 1, :] = jnp.full((1, 128), loss, jnp.float32)


_NC, _NS, _L = 2, 16, 16
_NW = _NC * _NS
_RPW = _N // _NW
_GN = 8
_NGR = _RPW // _GN
_NSLOT = _K + 1
_GROWS = 96


def _gat_sc_body(h_hbm, f_hbm, idx_hbm, out_hbm,
                 f_v, idx_v, gidx0_v, gidx1_v, nbr0_v, nbr1_v,
                 att_v, out_v, sem0, sem1):
    wid = lax.axis_index("s") * _NC + lax.axis_index("c")
    row0 = wid * _RPW
    pltpu.sync_copy(f_hbm, f_v)
    pltpu.sync_copy(idx_hbm.at[pl.ds(row0 * 16, _RPW * 16)], idx_v)
    lanes = lax.iota(jnp.int32, _L)

    def fire(g, gidx_v, nbr_v, sem):
        gbase = g * _GN
        nbase = row0 + gbase
        for k in range(_GROWS // _L):
            flat = lanes + k * _L
            n = lax.div(flat, _NSLOT)
            t = flat - n * _NSLOT
            n = jnp.minimum(n, _GN - 1)
            nb = plsc.load_gather(idx_v, [(gbase + n) * 16 + t])
            gidx_v[pl.ds(k * _L, _L)] = jnp.where(t == _K, nbase + n, nb)
        pltpu.async_copy(h_hbm.at[gidx_v], nbr_v, sem)

    def compute(g, gidx_v, nbr_v, sem):
        gbase = g * _GN
        nbase = row0 + gbase
        pltpu.make_async_copy(h_hbm.at[gidx_v], nbr_v, sem).wait()
        nloc = lanes & (_GN - 1)
        node = nbase + nloc
        for c in range(_NH):
            f1 = plsc.load_gather(f_v, [node * 8 + c])
            s = None
            ps = []
            for t in range(_NSLOT):
                if t == _K:
                    nb = node
                else:
                    nb = plsc.load_gather(idx_v, [(gbase + nloc) * 16 + t])
                f2 = plsc.load_gather(f_v, [nb * 8 + (_NH + c)])
                z = f1 + f2
                p = jnp.exp(jnp.maximum(z, _ALPHA * z))
                ps.append(p)
                s = p if s is None else s + p
            inv = 1.0 / s
            for t in range(_NSLOT):
                att_v[pl.ds((c * _NSLOT + t) * _L, _L)] = ps[t] * inv
        @plsc.parallel_loop(0, _GN, 1, unroll=2)
        def node_body(nl):
            for c in range(_NH):
                accs = [jnp.zeros((_L,), jnp.float32) for _ in range(_G // _L)]
                for t in range(_NSLOT):
                    a = plsc.load_gather(
                        att_v,
                        [jnp.full((_L,), (c * _NSLOT + t) * _L, jnp.int32) + nl])
                    row = nl * _NSLOT + t
                    for u in range(_G // _L):
                        accs[u] = accs[u] + a * nbr_v[row, pl.ds(c * _G + u * _L, _L)]
                for u in range(_G // _L):
                    o = accs[u]
                    out_v[nl, pl.ds(c * _G + u * _L, _L)] = (
                        jnp.where(o > 0.0, o, jnp.exp(o) - 1.0))
        pltpu.sync_copy(out_v, out_hbm.at[pl.ds(nbase, _GN)])

    fire(0, gidx0_v, nbr0_v, sem0)

    def pair(i, carry):
        fire(2 * i + 1, gidx1_v, nbr1_v, sem1)
        compute(2 * i, gidx0_v, nbr0_v, sem0)

        @pl.when(i < _NGR // 2 - 1)
        def _():
            fire(2 * i + 2, gidx0_v, nbr0_v, sem0)

        compute(2 * i + 1, gidx1_v, nbr1_v, sem1)
        return carry

    lax.fori_loop(0, _NGR // 2, pair, None)


_gat_sc = functools.partial(
    pl.kernel,
    out_type=jax.ShapeDtypeStruct((_N, _F), jnp.float32),
    mesh=plsc.VectorSubcoreMesh(core_axis_name="c", subcore_axis_name="s"),
    scratch_types=[
        pltpu.VMEM((_N * 8,), jnp.float32),
        pltpu.VMEM((_RPW * 16,), jnp.int32),
        pltpu.VMEM((_GROWS,), jnp.int32),
        pltpu.VMEM((_GROWS,), jnp.int32),
        pltpu.VMEM((_GROWS, _F), jnp.float32),
        pltpu.VMEM((_GROWS, _F), jnp.float32),
        pltpu.VMEM((_NH * _NSLOT * _L,), jnp.float32),
        pltpu.VMEM((_GN, _F), jnp.float32),
        pltpu.SemaphoreType.DMA,
        pltpu.SemaphoreType.DMA,
    ],
    compiler_params=pltpu.CompilerParams(needs_layout_passes=False),
)(_gat_sc_body)


def _build_A(a):
    a2 = a[:, :, 0]
    A = jnp.zeros((_F, 8), jnp.float32)
    for c in range(_NH):
        A = A.at[c * _G:(c + 1) * _G, c].set(a2[c, :_G])
        A = A.at[c * _G:(c + 1) * _G, _NH + c].set(a2[c, _G:])
    return A


def _hff(x, gat_W, gat_a, in_dim):
    Wcat = jnp.transpose(gat_W, (1, 0, 2)).reshape(in_dim, _F)
    A = _build_A(gat_a)
    return pl.pallas_call(
        _hff_body,
        grid=(4,),
        in_specs=[
            pl.BlockSpec((_N // 4, in_dim), lambda i: (i, 0)),
            pl.BlockSpec((in_dim, _F), lambda i: (0, 0)),
            pl.BlockSpec((_F, 8), lambda i: (0, 0)),
        ],
        out_specs=[
            pl.BlockSpec((_N // 4, _F), lambda i: (i, 0)),
            pl.BlockSpec((_N // 4, 8), lambda i: (i, 0)),
        ],
        out_shape=[
            jax.ShapeDtypeStruct((_N, _F), jnp.float32),
            jax.ShapeDtypeStruct((_N, 8), jnp.float32),
        ],
    )(x, Wcat, A)


def kernel(sinputs, tinputs, slabels, W0, b0, W1, b1,
           gat0_W, gat0_a, gat1_W, gat1_a, Wc, bc, Wd, bd):
    x_in = jnp.concatenate([sinputs.reshape(_D * _B, _IN), tinputs], axis=0)
    W0cat = jnp.transpose(gat0_W, (1, 0, 2)).reshape(_H2, _F)
    A0 = _build_A(gat0_a)
    h2, h_all0, f0 = pl.pallas_call(
        _mlp_body,
        grid=(4,),
        in_specs=[
            pl.BlockSpec((_N // 4, _IN), lambda i: (i, 0)),
            pl.BlockSpec((_IN, _H1), lambda i: (0, 0)),
            pl.BlockSpec((1, _H1), lambda i: (0, 0)),
            pl.BlockSpec((_H1, _H2), lambda i: (0, 0)),
            pl.BlockSpec((1, _H2), lambda i: (0, 0)),
            pl.BlockSpec((_H2, _F), lambda i: (0, 0)),
            pl.BlockSpec((_F, 8), lambda i: (0, 0)),
        ],
        out_specs=[
            pl.BlockSpec((_N // 4, _H2), lambda i: (i, 0)),
            pl.BlockSpec((_N // 4, _F), lambda i: (i, 0)),
            pl.BlockSpec((_N // 4, 8), lambda i: (i, 0)),
        ],
        out_shape=[
            jax.ShapeDtypeStruct((_N, _H2), jnp.float32),
            jax.ShapeDtypeStruct((_N, _F), jnp.float32),
            jax.ShapeDtypeStruct((_N, 8), jnp.float32),
        ],
    )(x_in, W0, b0.reshape(1, _H1), W1, b1.reshape(1, _H2), W0cat, A0)

    idx = pl.pallas_call(
        _topk_body,
        grid=(_N // _RB,),
        in_specs=[
            pl.BlockSpec((_RB, _H2), lambda i: (i, 0)),
            pl.BlockSpec((_N, _H2), lambda i: (0, 0)),
        ],
        out_specs=pl.BlockSpec((_RB, 16), lambda i: (i, 0)),
        out_shape=jax.ShapeDtypeStruct((_N, 16), jnp.int32),
    )(h2, h2)

    idxf = idx.reshape(-1)
    x1 = _gat_sc(h_all0, f0.reshape(-1), idxf)
    h_all1, f1_ = _hff(x1, gat1_W, gat1_a, _F)
    x2 = _gat_sc(h_all1, f1_.reshape(-1), idxf)

    lab = slabels.astype(jnp.int32)
    labt = lab.T
    lp, sd, td, tl = pl.pallas_call(
        _heads_body,
        out_shape=[
            jax.ShapeDtypeStruct((_D, _B, _NCLS), jnp.float32),
            jax.ShapeDtypeStruct((_D, _B, 2), jnp.float32),
            jax.ShapeDtypeStruct((_D, _B, 2), jnp.float32),
            jax.ShapeDtypeStruct((_D, 128), jnp.float32),
        ],
    )(x2, lab, labt, Wc, bc.reshape(1, _NCLS), Wd, bd)
    return lp, sd, td, tl[:, 0]


# speedup vs baseline: 1.0567x; 1.0567x over previous
import functools

import jax
import jax.numpy as jnp
from jax import lax
from jax.experimental import pallas as pl
from jax.experimental.pallas import tpu as pltpu
from jax.experimental.pallas import tpu_sc as plsc

_D = 3
_B = 512
_IN = 512
_H1, _H2 = 512, 256
_G = 128
_NH = 4
_K = 10
_ALPHA = 0.2
_MARGIN = 1.0
_NCLS = 10
_N = (_D + 1) * _B
_F = _NH * _G
_RB = 256
_NEG = -9e15


def _mlp_body(x_ref, w0_ref, b0_ref, w1_ref, b1_ref, o_ref):
    h = jnp.dot(x_ref[...], w0_ref[...], preferred_element_type=jnp.float32)
    h = jnp.maximum(h + b0_ref[...], 0.0)
    h = jnp.dot(h, w1_ref[...], preferred_element_type=jnp.float32)
    o_ref[...] = jnp.maximum(h + b1_ref[...], 0.0)


def _topk_body(hb_ref, hf_ref, idx_ref):
    i = pl.program_id(0)
    hb = hb_ref[...]
    hf = hf_ref[...]
    xxb = jnp.sum(hb * hb, axis=1, keepdims=True)
    xxf = jnp.sum(hf * hf, axis=1)
    g = jax.lax.dot_general(hb, hf, (((1,), (1,)), ((), ())),
                            preferred_element_type=jnp.float32)
    d2 = jnp.maximum(xxb + xxf[None, :] - 2.0 * g, 0.0)
    cols = jax.lax.broadcasted_iota(jnp.int32, d2.shape, 1)
    rows = jax.lax.broadcasted_iota(jnp.int32, d2.shape, 0) + i * _RB
    neg = jnp.where(rows == cols, -1e12, -d2)
    picks = []
    for _ in range(_K):
        m = jnp.max(neg, axis=1, keepdims=True)
        it = jnp.min(jnp.where(neg == m, cols, _N), axis=1, keepdims=True)
        picks.append(it)
        neg = jnp.where(cols == it, -jnp.float32(jnp.inf), neg)
    picks.append(jnp.zeros((_RB, 16 - _K), jnp.int32))
    idx_ref[...] = jnp.concatenate(picks, axis=1)


def _hff_body(x_ref, w_ref, a_ref, h_ref, f_ref):
    h = jnp.dot(x_ref[...], w_ref[...], preferred_element_type=jnp.float32)
    h_ref[...] = h
    f_ref[...] = jnp.dot(h, a_ref[...], preferred_element_type=jnp.float32)


def _log_softmax(x):
    m = jnp.max(x, axis=1, keepdims=True)
    return (x - m) - jnp.log(jnp.sum(jnp.exp(x - m), axis=1, keepdims=True))


def _heads_body(x_ref, lab_ref, labt_ref, wc_ref, bc_ref, wd_ref, bd_ref,
                lp_ref, sd_ref, td_ref, tl_ref):
    gt = x_ref[_D * _B:, :]
    rt = jnp.maximum(gt, 0.0)
    cols = jax.lax.broadcasted_iota(jnp.int32, (_B, _B), 1)
    rows = jax.lax.broadcasted_iota(jnp.int32, (_B, _B), 0)
    eye = rows == cols
    for d in range(_D):
        g = x_ref[d * _B:(d + 1) * _B, :]
        r = jnp.maximum(g, 0.0)
        lg = jnp.dot(r, wc_ref[...], preferred_element_type=jnp.float32) + bc_ref[...]
        lp_ref[d] = _log_softmax(lg)
        wd = wd_ref[d]
        bd = bd_ref[d:d + 1, :]
        sd_ref[d] = _log_softmax(
            jnp.dot(r, wd, preferred_element_type=jnp.float32) + bd)
        td_ref[d] = _log_softmax(
            jnp.dot(rt, wd, preferred_element_type=jnp.float32) + bd)
        nrm = jnp.sqrt(jnp.sum(g * g, axis=1, keepdims=True))
        gn = g / jnp.maximum(nrm, 1e-12)
        gram = jax.lax.dot_general(gn, gn, (((1,), (1,)), ((), ())),
                                   preferred_element_type=jnp.float32)
        diag = jnp.where(eye, gram, 0.0)
        xxc = jnp.sum(diag, axis=1, keepdims=True)
        xxr = jnp.sum(diag, axis=0, keepdims=True)
        dist = jnp.sqrt(jnp.maximum(xxc + xxr - 2.0 * gram, 0.0) + 1e-12)
        lr = lab_ref[d:d + 1, :]
        lc = labt_ref[:, d:d + 1]
        same = lc == lr
        pos_mask = same & (~eye)
        neg_mask = ~same
        pv = jnp.where(pos_mask, dist, -1.0)
        pm = jnp.max(pv, axis=1, keepdims=True)
        pidx = jnp.min(jnp.where(pv == pm, cols, _B), axis=1, keepdims=True)
        nv = jnp.where(neg_mask, dist, 1e12)
        nm = jnp.min(nv, axis=1, keepdims=True)
        nidx = jnp.min(jnp.where(nv == nm, cols, _B), axis=1, keepdims=True)
        pos_d = jnp.sum(jnp.where(cols == pidx, dist, 0.0), axis=1)
        neg_d = jnp.sum(jnp.where(cols == nidx, dist, 0.0), axis=1)
        hard = (neg_d - pos_d < _MARGIN).astype(jnp.float32)
        hinge = jnp.maximum(_MARGIN + pos_d - neg_d, 0.0)
        loss = jnp.sum(hinge * hard) / jnp.maximum(jnp.sum(hard), 1.0)
        tl_ref[d:d + 1, :] = jnp.full((1, 128), loss, jnp.float32)


_NC, _NS, _L = 2, 16, 16
_NW = _NC * _NS
_RPW = _N // _NW
_GN = 8
_NGR = _RPW // _GN
_NSLOT = _K + 1
_GROWS = 96


def _gat_sc_body(h_hbm, f_hbm, idx_hbm, out_hbm,
                 f_v, idx_v, gidx0_v, gidx1_v, nbr0_v, nbr1_v,
                 att_v, out_v, sem0, sem1):
    wid = lax.axis_index("s") * _NC + lax.axis_index("c")
    row0 = wid * _RPW
    pltpu.sync_copy(f_hbm, f_v)
    pltpu.sync_copy(idx_hbm.at[pl.ds(row0 * 16, _RPW * 16)], idx_v)
    lanes = lax.iota(jnp.int32, _L)

    def fire(g, gidx_v, nbr_v, sem):
        gbase = g * _GN
        nbase = row0 + gbase
        for k in range(_GROWS // _L):
            flat = lanes + k * _L
            n = lax.div(flat, _NSLOT)
            t = flat - n * _NSLOT
            n = jnp.minimum(n, _GN - 1)
            nb = plsc.load_gather(idx_v, [(gbase + n) * 16 + t])
            gidx_v[pl.ds(k * _L, _L)] = jnp.where(t == _K, nbase + n, nb)
        pltpu.async_copy(h_hbm.at[gidx_v], nbr_v, sem)

    def compute(g, gidx_v, nbr_v, sem):
        gbase = g * _GN
        nbase = row0 + gbase
        pltpu.make_async_copy(h_hbm.at[gidx_v], nbr_v, sem).wait()
        nloc = lanes & (_GN - 1)
        node = nbase + nloc
        for c in range(_NH):
            f1 = plsc.load_gather(f_v, [node * 8 + c])
            s = None
            ps = []
            for t in range(_NSLOT):
                if t == _K:
                    nb = node
                else:
                    nb = plsc.load_gather(idx_v, [(gbase + nloc) * 16 + t])
                f2 = plsc.load_gather(f_v, [nb * 8 + (_NH + c)])
                z = f1 + f2
                p = jnp.exp(jnp.maximum(z, _ALPHA * z))
                ps.append(p)
                s = p if s is None else s + p
            inv = 1.0 / s
            for t in range(_NSLOT):
                att_v[pl.ds((c * _NSLOT + t) * _L, _L)] = ps[t] * inv
        def node_body(nl, carry2):
            for c in range(_NH):
                accs = [jnp.zeros((_L,), jnp.float32) for _ in range(_G // _L)]
                for t in range(_NSLOT):
                    a = plsc.load_gather(
                        att_v,
                        [jnp.full((_L,), (c * _NSLOT + t) * _L, jnp.int32) + nl])
                    row = nl * _NSLOT + t
                    for u in range(_G // _L):
                        accs[u] = accs[u] + a * nbr_v[row, pl.ds(c * _G + u * _L, _L)]
                for u in range(_G // _L):
                    o = accs[u]
                    out_v[nl, pl.ds(c * _G + u * _L, _L)] = (
                        jnp.where(o > 0.0, o, jnp.exp(o) - 1.0))
            return carry2
        lax.fori_loop(0, _GN, node_body, None)
        pltpu.sync_copy(out_v, out_hbm.at[pl.ds(nbase, _GN)])

    fire(0, gidx0_v, nbr0_v, sem0)

    def pair(i, carry):
        fire(2 * i + 1, gidx1_v, nbr1_v, sem1)
        compute(2 * i, gidx0_v, nbr0_v, sem0)

        @pl.when(i < _NGR // 2 - 1)
        def _():
            fire(2 * i + 2, gidx0_v, nbr0_v, sem0)

        compute(2 * i + 1, gidx1_v, nbr1_v, sem1)
        return carry

    lax.fori_loop(0, _NGR // 2, pair, None)


_gat_sc = functools.partial(
    pl.kernel,
    out_type=jax.ShapeDtypeStruct((_N, _F), jnp.float32),
    mesh=plsc.VectorSubcoreMesh(core_axis_name="c", subcore_axis_name="s"),
    scratch_types=[
        pltpu.VMEM((_N * 8,), jnp.float32),
        pltpu.VMEM((_RPW * 16,), jnp.int32),
        pltpu.VMEM((_GROWS,), jnp.int32),
        pltpu.VMEM((_GROWS,), jnp.int32),
        pltpu.VMEM((_GROWS, _F), jnp.float32),
        pltpu.VMEM((_GROWS, _F), jnp.float32),
        pltpu.VMEM((_NH * _NSLOT * _L,), jnp.float32),
        pltpu.VMEM((_GN, _F), jnp.float32),
        pltpu.SemaphoreType.DMA,
        pltpu.SemaphoreType.DMA,
    ],
    compiler_params=pltpu.CompilerParams(needs_layout_passes=False),
)(_gat_sc_body)


def _build_A(a):
    a2 = a[:, :, 0]
    A = jnp.zeros((_F, 8), jnp.float32)
    for c in range(_NH):
        A = A.at[c * _G:(c + 1) * _G, c].set(a2[c, :_G])
        A = A.at[c * _G:(c + 1) * _G, _NH + c].set(a2[c, _G:])
    return A


def _hff(x, gat_W, gat_a, in_dim):
    Wcat = jnp.transpose(gat_W, (1, 0, 2)).reshape(in_dim, _F)
    A = _build_A(gat_a)
    return pl.pallas_call(
        _hff_body,
        grid=(4,),
        in_specs=[
            pl.BlockSpec((_N // 4, in_dim), lambda i: (i, 0)),
            pl.BlockSpec((in_dim, _F), lambda i: (0, 0)),
            pl.BlockSpec((_F, 8), lambda i: (0, 0)),
        ],
        out_specs=[
            pl.BlockSpec((_N // 4, _F), lambda i: (i, 0)),
            pl.BlockSpec((_N // 4, 8), lambda i: (i, 0)),
        ],
        out_shape=[
            jax.ShapeDtypeStruct((_N, _F), jnp.float32),
            jax.ShapeDtypeStruct((_N, 8), jnp.float32),
        ],
    )(x, Wcat, A)


def kernel(sinputs, tinputs, slabels, W0, b0, W1, b1,
           gat0_W, gat0_a, gat1_W, gat1_a, Wc, bc, Wd, bd):
    x_in = jnp.concatenate([sinputs.reshape(_D * _B, _IN), tinputs], axis=0)
    h2 = pl.pallas_call(
        _mlp_body,
        grid=(4,),
        in_specs=[
            pl.BlockSpec((_N // 4, _IN), lambda i: (i, 0)),
            pl.BlockSpec((_IN, _H1), lambda i: (0, 0)),
            pl.BlockSpec((1, _H1), lambda i: (0, 0)),
            pl.BlockSpec((_H1, _H2), lambda i: (0, 0)),
            pl.BlockSpec((1, _H2), lambda i: (0, 0)),
        ],
        out_specs=pl.BlockSpec((_N // 4, _H2), lambda i: (i, 0)),
        out_shape=jax.ShapeDtypeStruct((_N, _H2), jnp.float32),
    )(x_in, W0, b0.reshape(1, _H1), W1, b1.reshape(1, _H2))

    idx = pl.pallas_call(
        _topk_body,
        grid=(_N // _RB,),
        in_specs=[
            pl.BlockSpec((_RB, _H2), lambda i: (i, 0)),
            pl.BlockSpec((_N, _H2), lambda i: (0, 0)),
        ],
        out_specs=pl.BlockSpec((_RB, 16), lambda i: (i, 0)),
        out_shape=jax.ShapeDtypeStruct((_N, 16), jnp.int32),
    )(h2, h2)

    idxf = idx.reshape(-1)
    h_all0, f0 = _hff(h2, gat0_W, gat0_a, _H2)
    x1 = _gat_sc(h_all0, f0.reshape(-1), idxf)
    h_all1, f1_ = _hff(x1, gat1_W, gat1_a, _F)
    x2 = _gat_sc(h_all1, f1_.reshape(-1), idxf)

    lab = slabels.astype(jnp.int32)
    labt = lab.T
    lp, sd, td, tl = pl.pallas_call(
        _heads_body,
        out_shape=[
            jax.ShapeDtypeStruct((_D, _B, _NCLS), jnp.float32),
            jax.ShapeDtypeStruct((_D, _B, 2), jnp.float32),
            jax.ShapeDtypeStruct((_D, _B, 2), jnp.float32),
            jax.ShapeDtypeStruct((_D, 128), jnp.float32),
        ],
    )(x2, lab, labt, Wc, bc.reshape(1, _NCLS), Wd, bd)
    return lp, sd, td, tl[:, 0]


# speedup vs baseline: 1.1142x; 1.0545x over previous
import functools

import jax
import jax.numpy as jnp
from jax import lax
from jax.experimental import pallas as pl
from jax.experimental.pallas import tpu as pltpu
from jax.experimental.pallas import tpu_sc as plsc

_D = 3
_B = 512
_IN = 512
_H1, _H2 = 512, 256
_G = 128
_NH = 4
_K = 10
_ALPHA = 0.2
_MARGIN = 1.0
_NCLS = 10
_N = (_D + 1) * _B
_F = _NH * _G
_RB = 256


def _mlp_body(x_ref, w0_ref, b0_ref, w1_ref, b1_ref, o_ref):
    h = jnp.dot(x_ref[...], w0_ref[...], preferred_element_type=jnp.float32)
    h = jnp.maximum(h + b0_ref[...], 0.0)
    h = jnp.dot(h, w1_ref[...], preferred_element_type=jnp.float32)
    o_ref[...] = jnp.maximum(h + b1_ref[...], 0.0)


def _topk_body(hb_ref, hf_ref, idx_ref):
    i = pl.program_id(0)
    hb = hb_ref[...]
    hf = hf_ref[...]
    xxb = jnp.sum(hb * hb, axis=1, keepdims=True)
    xxf = jnp.sum(hf * hf, axis=1)
    g = jax.lax.dot_general(hb, hf, (((1,), (1,)), ((), ())),
                            preferred_element_type=jnp.float32)
    d2 = jnp.maximum(xxb + xxf[None, :] - 2.0 * g, 0.0)
    cols = jax.lax.broadcasted_iota(jnp.int32, d2.shape, 1).astype(jnp.float32)
    rows = (jax.lax.broadcasted_iota(jnp.int32, d2.shape, 0) + i * _RB).astype(
        jnp.float32)
    neg = jnp.where(rows == cols, -1e12, -d2)
    picks = []
    for _ in range(_K):
        m = jnp.max(neg, axis=1, keepdims=True)
        it = jnp.min(jnp.where(neg == m, cols, jnp.float32(_N)),
                     axis=1, keepdims=True)
        picks.append(it.astype(jnp.int32))
        neg = jnp.where(cols == it, -jnp.float32(jnp.inf), neg)
    picks.append(jnp.zeros((_RB, 16 - _K), jnp.int32))
    idx_ref[...] = jnp.concatenate(picks, axis=1)


def _hff_body(x_ref, w_ref, a_ref, h_ref, f_ref):
    h = jnp.dot(x_ref[...], w_ref[...], preferred_element_type=jnp.float32)
    h_ref[...] = h
    f_ref[...] = jnp.dot(h, a_ref[...], preferred_element_type=jnp.float32)


def _log_softmax(x):
    m = jnp.max(x, axis=1, keepdims=True)
    return (x - m) - jnp.log(jnp.sum(jnp.exp(x - m), axis=1, keepdims=True))


def _heads_body(x_ref, lab_ref, labt_ref, wc_ref, bc_ref, wd_ref, bd_ref,
                lp_ref, sd_ref, td_ref, tl_ref):
    gt = x_ref[_D * _B:, :]
    rt = jnp.maximum(gt, 0.0)
    cols = jax.lax.broadcasted_iota(jnp.int32, (_B, _B), 1)
    rows = jax.lax.broadcasted_iota(jnp.int32, (_B, _B), 0)
    eye = rows == cols
    for d in range(_D):
        g = x_ref[d * _B:(d + 1) * _B, :]
        r = jnp.maximum(g, 0.0)
        lg = jnp.dot(r, wc_ref[...], preferred_element_type=jnp.float32) + bc_ref[...]
        lp_ref[d] = _log_softmax(lg)
        wd = wd_ref[d]
        bd = bd_ref[d:d + 1, :]
        sd_ref[d] = _log_softmax(
            jnp.dot(r, wd, preferred_element_type=jnp.float32) + bd)
        td_ref[d] = _log_softmax(
            jnp.dot(rt, wd, preferred_element_type=jnp.float32) + bd)
        nrm = jnp.sqrt(jnp.sum(g * g, axis=1, keepdims=True))
        gn = g / jnp.maximum(nrm, 1e-12)
        gram = jax.lax.dot_general(gn, gn, (((1,), (1,)), ((), ())),
                                   preferred_element_type=jnp.float32)
        diag = jnp.where(eye, gram, 0.0)
        xxc = jnp.sum(diag, axis=1, keepdims=True)
        xxr = jnp.sum(diag, axis=0, keepdims=True)
        dist = jnp.sqrt(jnp.maximum(xxc + xxr - 2.0 * gram, 0.0) + 1e-12)
        lr = lab_ref[d:d + 1, :]
        lc = labt_ref[:, d:d + 1]
        same = lc == lr
        pos_mask = same & (~eye)
        neg_mask = ~same
        pv = jnp.where(pos_mask, dist, -1.0)
        pm = jnp.max(pv, axis=1, keepdims=True)
        pidx = jnp.min(jnp.where(pv == pm, cols, _B), axis=1, keepdims=True)
        nv = jnp.where(neg_mask, dist, 1e12)
        nm = jnp.min(nv, axis=1, keepdims=True)
        nidx = jnp.min(jnp.where(nv == nm, cols, _B), axis=1, keepdims=True)
        pos_d = jnp.sum(jnp.where(cols == pidx, dist, 0.0), axis=1)
        neg_d = jnp.sum(jnp.where(cols == nidx, dist, 0.0), axis=1)
        hard = (neg_d - pos_d < _MARGIN).astype(jnp.float32)
        hinge = jnp.maximum(_MARGIN + pos_d - neg_d, 0.0)
        loss = jnp.sum(hinge * hard) / jnp.maximum(jnp.sum(hard), 1.0)
        tl_ref[d:d + 1, :] = jnp.full((1, 128), loss, jnp.float32)


_NC, _NS, _L = 2, 16, 16
_NW = _NC * _NS
_RPW = _N // _NW
_GN = 8
_NGR = _RPW // _GN
_NSLOT = _K + 1
_GROWS = 96


def _gat_sc_body(h_hbm, f_hbm, idx_hbm, out_hbm,
                 f_v, idx_v, gidx0_v, gidx1_v, nbr0_v, nbr1_v,
                 att_v, out_v, sem0, sem1):
    wid = lax.axis_index("s") * _NC + lax.axis_index("c")
    row0 = wid * _RPW
    pltpu.sync_copy(f_hbm, f_v)
    pltpu.sync_copy(idx_hbm.at[pl.ds(row0 * 16, _RPW * 16)], idx_v)
    lanes = lax.iota(jnp.int32, _L)

    def fire(g, gidx_v, nbr_v, sem):
        gbase = g * _GN
        nbase = row0 + gbase
        for k in range(_GROWS // _L):
            flat = lanes + k * _L
            n = lax.div(flat, _NSLOT)
            t = flat - n * _NSLOT
            n = jnp.minimum(n, _GN - 1)
            nb = plsc.load_gather(idx_v, [(gbase + n) * 16 + t])
            gidx_v[pl.ds(k * _L, _L)] = jnp.where(t == _K, nbase + n, nb)
        pltpu.async_copy(h_hbm.at[gidx_v], nbr_v, sem)

    def compute(g, gidx_v, nbr_v, sem):
        gbase = g * _GN
        nbase = row0 + gbase
        pltpu.make_async_copy(h_hbm.at[gidx_v], nbr_v, sem).wait()
        nloc = lanes & (_GN - 1)
        node = nbase + nloc
        for c in range(_NH):
            f1 = plsc.load_gather(f_v, [node * 8 + c])
            s = None
            ps = []
            for t in range(_NSLOT):
                if t == _K:
                    nb = node
                else:
                    nb = plsc.load_gather(idx_v, [(gbase + nloc) * 16 + t])
                f2 = plsc.load_gather(f_v, [nb * 8 + (_NH + c)])
                z = f1 + f2
                p = jnp.exp(jnp.maximum(z, _ALPHA * z))
                ps.append(p)
                s = p if s is None else s + p
            inv = 1.0 / s
            for t in range(_NSLOT):
                att_v[pl.ds((c * _NSLOT + t) * _L, _L)] = ps[t] * inv
        def node_body(nl, carry2):
            for c in range(_NH):
                accs = [jnp.zeros((_L,), jnp.float32) for _ in range(_G // _L)]
                for t in range(_NSLOT):
                    a = plsc.load_gather(
                        att_v,
                        [jnp.full((_L,), (c * _NSLOT + t) * _L, jnp.int32) + nl])
                    row = nl * _NSLOT + t
                    for u in range(_G // _L):
                        accs[u] = accs[u] + a * nbr_v[row, pl.ds(c * _G + u * _L, _L)]
                for u in range(_G // _L):
                    o = accs[u]
                    out_v[nl, pl.ds(c * _G + u * _L, _L)] = (
                        jnp.where(o > 0.0, o, jnp.exp(o) - 1.0))
            return carry2
        lax.fori_loop(0, _GN, node_body, None)
        pltpu.sync_copy(out_v, out_hbm.at[pl.ds(nbase, _GN)])

    fire(0, gidx0_v, nbr0_v, sem0)

    def pair(i, carry):
        fire(2 * i + 1, gidx1_v, nbr1_v, sem1)
        compute(2 * i, gidx0_v, nbr0_v, sem0)

        @pl.when(i < _NGR // 2 - 1)
        def _():
            fire(2 * i + 2, gidx0_v, nbr0_v, sem0)

        compute(2 * i + 1, gidx1_v, nbr1_v, sem1)
        return carry

    lax.fori_loop(0, _NGR // 2, pair, None)


_gat_sc = functools.partial(
    pl.kernel,
    out_type=jax.ShapeDtypeStruct((_N, _F), jnp.float32),
    mesh=plsc.VectorSubcoreMesh(core_axis_name="c", subcore_axis_name="s"),
    scratch_types=[
        pltpu.VMEM((_N * 8,), jnp.float32),
        pltpu.VMEM((_RPW * 16,), jnp.int32),
        pltpu.VMEM((_GROWS,), jnp.int32),
        pltpu.VMEM((_GROWS,), jnp.int32),
        pltpu.VMEM((_GROWS, _F), jnp.float32),
        pltpu.VMEM((_GROWS, _F), jnp.float32),
        pltpu.VMEM((_NH * _NSLOT * _L,), jnp.float32),
        pltpu.VMEM((_GN, _F), jnp.float32),
        pltpu.SemaphoreType.DMA,
        pltpu.SemaphoreType.DMA,
    ],
    compiler_params=pltpu.CompilerParams(needs_layout_passes=False),
)(_gat_sc_body)


def _build_A(a):
    a2 = a[:, :, 0]
    A = jnp.zeros((_F, 8), jnp.float32)
    for c in range(_NH):
        A = A.at[c * _G:(c + 1) * _G, c].set(a2[c, :_G])
        A = A.at[c * _G:(c + 1) * _G, _NH + c].set(a2[c, _G:])
    return A


def _hff(x, gat_W, gat_a, in_dim):
    Wcat = jnp.transpose(gat_W, (1, 0, 2)).reshape(in_dim, _F)
    A = _build_A(gat_a)
    return pl.pallas_call(
        _hff_body,
        grid=(4,),
        in_specs=[
            pl.BlockSpec((_N // 4, in_dim), lambda i: (i, 0)),
            pl.BlockSpec((in_dim, _F), lambda i: (0, 0)),
            pl.BlockSpec((_F, 8), lambda i: (0, 0)),
        ],
        out_specs=[
            pl.BlockSpec((_N // 4, _F), lambda i: (i, 0)),
            pl.BlockSpec((_N // 4, 8), lambda i: (i, 0)),
        ],
        out_shape=[
            jax.ShapeDtypeStruct((_N, _F), jnp.float32),
            jax.ShapeDtypeStruct((_N, 8), jnp.float32),
        ],
    )(x, Wcat, A)


def kernel(sinputs, tinputs, slabels, W0, b0, W1, b1,
           gat0_W, gat0_a, gat1_W, gat1_a, Wc, bc, Wd, bd):
    x_in = jnp.concatenate([sinputs.reshape(_D * _B, _IN), tinputs], axis=0)
    h2 = pl.pallas_call(
        _mlp_body,
        grid=(4,),
        in_specs=[
            pl.BlockSpec((_N // 4, _IN), lambda i: (i, 0)),
            pl.BlockSpec((_IN, _H1), lambda i: (0, 0)),
            pl.BlockSpec((1, _H1), lambda i: (0, 0)),
            pl.BlockSpec((_H1, _H2), lambda i: (0, 0)),
            pl.BlockSpec((1, _H2), lambda i: (0, 0)),
        ],
        out_specs=pl.BlockSpec((_N // 4, _H2), lambda i: (i, 0)),
        out_shape=jax.ShapeDtypeStruct((_N, _H2), jnp.float32),
    )(x_in, W0, b0.reshape(1, _H1), W1, b1.reshape(1, _H2))

    idx = pl.pallas_call(
        _topk_body,
        grid=(_N // _RB,),
        in_specs=[
            pl.BlockSpec((_RB, _H2), lambda i: (i, 0)),
            pl.BlockSpec((_N, _H2), lambda i: (0, 0)),
        ],
        out_specs=pl.BlockSpec((_RB, 16), lambda i: (i, 0)),
        out_shape=jax.ShapeDtypeStruct((_N, 16), jnp.int32),
    )(h2, h2)

    idxf = idx.reshape(-1)
    h_all0, f0 = _hff(h2, gat0_W, gat0_a, _H2)
    x1 = _gat_sc(h_all0, f0.reshape(-1), idxf)
    h_all1, f1_ = _hff(x1, gat1_W, gat1_a, _F)
    x2 = _gat_sc(h_all1, f1_.reshape(-1), idxf)

    lab = slabels.astype(jnp.int32)
    labt = lab.T
    lp, sd, td, tl = pl.pallas_call(
        _heads_body,
        out_shape=[
            jax.ShapeDtypeStruct((_D, _B, _NCLS), jnp.float32),
            jax.ShapeDtypeStruct((_D, _B, 2), jnp.float32),
            jax.ShapeDtypeStruct((_D, _B, 2), jnp.float32),
            jax.ShapeDtypeStruct((_D, 128), jnp.float32),
        ],
    )(x2, lab, labt, Wc, bc.reshape(1, _NCLS), Wd, bd)
    return lp, sd, td, tl[:, 0]


# speedup vs baseline: 1.1164x; 1.0019x over previous
import functools

import jax
import jax.numpy as jnp
from jax import lax
from jax.experimental import pallas as pl
from jax.experimental.pallas import tpu as pltpu
from jax.experimental.pallas import tpu_sc as plsc

_D = 3
_B = 512
_IN = 512
_H1, _H2 = 512, 256
_G = 128
_NH = 4
_K = 10
_ALPHA = 0.2
_MARGIN = 1.0
_NCLS = 10
_N = (_D + 1) * _B
_F = _NH * _G
_RB = 256


def _mlp_body(x_ref, w0_ref, b0_ref, w1_ref, b1_ref, o_ref):
    h = jnp.dot(x_ref[...], w0_ref[...], preferred_element_type=jnp.float32)
    h = jnp.maximum(h + b0_ref[...], 0.0)
    h = jnp.dot(h, w1_ref[...], preferred_element_type=jnp.float32)
    o_ref[...] = jnp.maximum(h + b1_ref[...], 0.0)


def _topk_body(hb_ref, hf_ref, idx_ref):
    i = pl.program_id(0)
    hb = hb_ref[...]
    hf = hf_ref[...]
    xxb = jnp.sum(hb * hb, axis=1, keepdims=True)
    xxf = jnp.sum(hf * hf, axis=1)
    g = jax.lax.dot_general(hb, hf, (((1,), (1,)), ((), ())),
                            preferred_element_type=jnp.float32)
    d2 = jnp.maximum(xxb + xxf[None, :] - 2.0 * g, 0.0)
    cols = jax.lax.broadcasted_iota(jnp.int32, d2.shape, 1).astype(jnp.float32)
    rows = (jax.lax.broadcasted_iota(jnp.int32, d2.shape, 0) + i * _RB).astype(
        jnp.float32)
    neg = jnp.where(rows == cols, -1e12, -d2)
    picks = []
    for _ in range(_K):
        m = jnp.max(neg, axis=1, keepdims=True)
        it = jnp.min(jnp.where(neg == m, cols, jnp.float32(_N)),
                     axis=1, keepdims=True)
        picks.append(it.astype(jnp.int32))
        neg = jnp.where(cols == it, -jnp.float32(jnp.inf), neg)
    picks.append(jnp.zeros((_RB, 16 - _K), jnp.int32))
    idx_ref[...] = jnp.concatenate(picks, axis=1)


def _hff_body(x_ref, w_ref, a_ref, h_ref, f_ref):
    h = jnp.dot(x_ref[...], w_ref[...], preferred_element_type=jnp.float32)
    h_ref[...] = h
    f_ref[...] = jnp.dot(h, a_ref[...], preferred_element_type=jnp.float32)


def _log_softmax(x):
    m = jnp.max(x, axis=1, keepdims=True)
    return (x - m) - jnp.log(jnp.sum(jnp.exp(x - m), axis=1, keepdims=True))


def _heads_body(x_ref, lab_ref, labt_ref, wc_ref, bc_ref, wd_ref, bd_ref,
                lp_ref, sd_ref, td_ref, tl_ref):
    gt = x_ref[_D * _B:, :]
    rt = jnp.maximum(gt, 0.0)
    cols = jax.lax.broadcasted_iota(jnp.int32, (_B, _B), 1).astype(jnp.float32)
    rows = jax.lax.broadcasted_iota(jnp.int32, (_B, _B), 0).astype(jnp.float32)
    eye = rows == cols
    for d in range(_D):
        g = x_ref[d * _B:(d + 1) * _B, :]
        r = jnp.maximum(g, 0.0)
        lg = jnp.dot(r, wc_ref[...], preferred_element_type=jnp.float32) + bc_ref[...]
        lp_ref[d] = _log_softmax(lg)
        wd = wd_ref[d]
        bd = bd_ref[d:d + 1, :]
        sd_ref[d] = _log_softmax(
            jnp.dot(r, wd, preferred_element_type=jnp.float32) + bd)
        td_ref[d] = _log_softmax(
            jnp.dot(rt, wd, preferred_element_type=jnp.float32) + bd)
        nrm = jnp.sqrt(jnp.sum(g * g, axis=1, keepdims=True))
        gn = g / jnp.maximum(nrm, 1e-12)
        gram = jax.lax.dot_general(gn, gn, (((1,), (1,)), ((), ())),
                                   preferred_element_type=jnp.float32)
        diag = jnp.where(eye, gram, 0.0)
        xxc = jnp.sum(diag, axis=1, keepdims=True)
        xxr = jnp.sum(diag, axis=0, keepdims=True)
        dist = jnp.sqrt(jnp.maximum(xxc + xxr - 2.0 * gram, 0.0) + 1e-12)
        lr = lab_ref[d:d + 1, :]
        lc = labt_ref[:, d:d + 1]
        same = lc == lr
        pos_mask = same & (~eye)
        neg_mask = ~same
        pv = jnp.where(pos_mask, dist, -1.0)
        pm = jnp.max(pv, axis=1, keepdims=True)
        pidx = jnp.min(jnp.where(pv == pm, cols, jnp.float32(_B)),
                       axis=1, keepdims=True)
        nv = jnp.where(neg_mask, dist, 1e12)
        nm = jnp.min(nv, axis=1, keepdims=True)
        nidx = jnp.min(jnp.where(nv == nm, cols, jnp.float32(_B)),
                       axis=1, keepdims=True)
        pos_d = jnp.sum(jnp.where(cols == pidx, dist, 0.0), axis=1)
        neg_d = jnp.sum(jnp.where(cols == nidx, dist, 0.0), axis=1)
        hard = (neg_d - pos_d < _MARGIN).astype(jnp.float32)
        hinge = jnp.maximum(_MARGIN + pos_d - neg_d, 0.0)
        loss = jnp.sum(hinge * hard) / jnp.maximum(jnp.sum(hard), 1.0)
        tl_ref[d:d + 1, :] = jnp.full((1, 128), loss, jnp.float32)


_NC, _NS, _L = 2, 16, 16
_NW = _NC * _NS
_RPW = _N // _NW
_GN = 8
_NGR = _RPW // _GN
_NSLOT = _K + 1
_GROWS = 96


def _gat_sc_body(h_hbm, f_hbm, idx_hbm, out_hbm,
                 f_v, idx_v, gidx0_v, gidx1_v, nbr0_v, nbr1_v,
                 att_v, out_v, sem0, sem1):
    wid = lax.axis_index("s") * _NC + lax.axis_index("c")
    row0 = wid * _RPW
    pltpu.sync_copy(f_hbm, f_v)
    pltpu.sync_copy(idx_hbm.at[pl.ds(row0 * 16, _RPW * 16)], idx_v)
    lanes = lax.iota(jnp.int32, _L)

    def fire(g, gidx_v, nbr_v, sem):
        gbase = g * _GN
        nbase = row0 + gbase
        for k in range(_GROWS // _L):
            flat = lanes + k * _L
            n = lax.div(flat, _NSLOT)
            t = flat - n * _NSLOT
            n = jnp.minimum(n, _GN - 1)
            nb = plsc.load_gather(idx_v, [(gbase + n) * 16 + t])
            gidx_v[pl.ds(k * _L, _L)] = jnp.where(t == _K, nbase + n, nb)
        pltpu.async_copy(h_hbm.at[gidx_v], nbr_v, sem)

    def compute(g, gidx_v, nbr_v, sem):
        gbase = g * _GN
        nbase = row0 + gbase
        pltpu.make_async_copy(h_hbm.at[gidx_v], nbr_v, sem).wait()
        nloc = lanes & (_GN - 1)
        node = nbase + nloc
        for c in range(_NH):
            f1 = plsc.load_gather(f_v, [node * 8 + c])
            s = None
            ps = []
            for t in range(_NSLOT):
                if t == _K:
                    nb = node
                else:
                    nb = plsc.load_gather(idx_v, [(gbase + nloc) * 16 + t])
                f2 = plsc.load_gather(f_v, [nb * 8 + (_NH + c)])
                z = f1 + f2
                p = jnp.exp(jnp.maximum(z, _ALPHA * z))
                ps.append(p)
                s = p if s is None else s + p
            inv = 1.0 / s
            for t in range(_NSLOT):
                att_v[pl.ds((c * _NSLOT + t) * _L, _L)] = ps[t] * inv
        def node_body(nl, carry2):
            for c in range(_NH):
                accs = [jnp.zeros((_L,), jnp.float32) for _ in range(_G // _L)]
                for t in range(_NSLOT):
                    a = plsc.load_gather(
                        att_v,
                        [jnp.full((_L,), (c * _NSLOT + t) * _L, jnp.int32) + nl])
                    row = nl * _NSLOT + t
                    for u in range(_G // _L):
                        accs[u] = accs[u] + a * nbr_v[row, pl.ds(c * _G + u * _L, _L)]
                for u in range(_G // _L):
                    o = accs[u]
                    out_v[nl, pl.ds(c * _G + u * _L, _L)] = (
                        jnp.where(o > 0.0, o, jnp.exp(o) - 1.0))
            return carry2
        lax.fori_loop(0, _GN, node_body, None)
        pltpu.sync_copy(out_v, out_hbm.at[pl.ds(nbase, _GN)])

    fire(0, gidx0_v, nbr0_v, sem0)

    def pair(i, carry):
        fire(2 * i + 1, gidx1_v, nbr1_v, sem1)
        compute(2 * i, gidx0_v, nbr0_v, sem0)

        @pl.when(i < _NGR // 2 - 1)
        def _():
            fire(2 * i + 2, gidx0_v, nbr0_v, sem0)

        compute(2 * i + 1, gidx1_v, nbr1_v, sem1)
        return carry

    lax.fori_loop(0, _NGR // 2, pair, None)


_gat_sc = functools.partial(
    pl.kernel,
    out_type=jax.ShapeDtypeStruct((_N, _F), jnp.float32),
    mesh=plsc.VectorSubcoreMesh(core_axis_name="c", subcore_axis_name="s"),
    scratch_types=[
        pltpu.VMEM((_N * 8,), jnp.float32),
        pltpu.VMEM((_RPW * 16,), jnp.int32),
        pltpu.VMEM((_GROWS,), jnp.int32),
        pltpu.VMEM((_GROWS,), jnp.int32),
        pltpu.VMEM((_GROWS, _F), jnp.float32),
        pltpu.VMEM((_GROWS, _F), jnp.float32),
        pltpu.VMEM((_NH * _NSLOT * _L,), jnp.float32),
        pltpu.VMEM((_GN, _F), jnp.float32),
        pltpu.SemaphoreType.DMA,
        pltpu.SemaphoreType.DMA,
    ],
    compiler_params=pltpu.CompilerParams(needs_layout_passes=False),
)(_gat_sc_body)


def _build_A(a):
    a2 = a[:, :, 0]
    A = jnp.zeros((_F, 8), jnp.float32)
    for c in range(_NH):
        A = A.at[c * _G:(c + 1) * _G, c].set(a2[c, :_G])
        A = A.at[c * _G:(c + 1) * _G, _NH + c].set(a2[c, _G:])
    return A


def _hff(x, gat_W, gat_a, in_dim):
    Wcat = jnp.transpose(gat_W, (1, 0, 2)).reshape(in_dim, _F)
    A = _build_A(gat_a)
    return pl.pallas_call(
        _hff_body,
        grid=(4,),
        in_specs=[
            pl.BlockSpec((_N // 4, in_dim), lambda i: (i, 0)),
            pl.BlockSpec((in_dim, _F), lambda i: (0, 0)),
            pl.BlockSpec((_F, 8), lambda i: (0, 0)),
        ],
        out_specs=[
            pl.BlockSpec((_N // 4, _F), lambda i: (i, 0)),
            pl.BlockSpec((_N // 4, 8), lambda i: (i, 0)),
        ],
        out_shape=[
            jax.ShapeDtypeStruct((_N, _F), jnp.float32),
            jax.ShapeDtypeStruct((_N, 8), jnp.float32),
        ],
    )(x, Wcat, A)


def kernel(sinputs, tinputs, slabels, W0, b0, W1, b1,
           gat0_W, gat0_a, gat1_W, gat1_a, Wc, bc, Wd, bd):
    x_in = jnp.concatenate([sinputs.reshape(_D * _B, _IN), tinputs], axis=0)
    h2 = pl.pallas_call(
        _mlp_body,
        grid=(4,),
        in_specs=[
            pl.BlockSpec((_N // 4, _IN), lambda i: (i, 0)),
            pl.BlockSpec((_IN, _H1), lambda i: (0, 0)),
            pl.BlockSpec((1, _H1), lambda i: (0, 0)),
            pl.BlockSpec((_H1, _H2), lambda i: (0, 0)),
            pl.BlockSpec((1, _H2), lambda i: (0, 0)),
        ],
        out_specs=pl.BlockSpec((_N // 4, _H2), lambda i: (i, 0)),
        out_shape=jax.ShapeDtypeStruct((_N, _H2), jnp.float32),
    )(x_in, W0, b0.reshape(1, _H1), W1, b1.reshape(1, _H2))

    idx = pl.pallas_call(
        _topk_body,
        grid=(_N // _RB,),
        in_specs=[
            pl.BlockSpec((_RB, _H2), lambda i: (i, 0)),
            pl.BlockSpec((_N, _H2), lambda i: (0, 0)),
        ],
        out_specs=pl.BlockSpec((_RB, 16), lambda i: (i, 0)),
        out_shape=jax.ShapeDtypeStruct((_N, 16), jnp.int32),
    )(h2, h2)

    idxf = idx.reshape(-1)
    h_all0, f0 = _hff(h2, gat0_W, gat0_a, _H2)
    x1 = _gat_sc(h_all0, f0.reshape(-1), idxf)
    h_all1, f1_ = _hff(x1, gat1_W, gat1_a, _F)
    x2 = _gat_sc(h_all1, f1_.reshape(-1), idxf)

    lab = slabels.astype(jnp.int32)
    labt = lab.T
    lp, sd, td, tl = pl.pallas_call(
        _heads_body,
        out_shape=[
            jax.ShapeDtypeStruct((_D, _B, _NCLS), jnp.float32),
            jax.ShapeDtypeStruct((_D, _B, 2), jnp.float32),
            jax.ShapeDtypeStruct((_D, _B, 2), jnp.float32),
            jax.ShapeDtypeStruct((_D, 128), jnp.float32),
        ],
    )(x2, lab, labt, Wc, bc.reshape(1, _NCLS), Wd, bd)
    return lp, sd, td, tl[:, 0]
